# initial kernel scaffold (unmeasured)
import jax
import jax.numpy as jnp
from jax import lax
from jax.experimental import pallas as pl
from jax.experimental.pallas import tpu as pltpu

B, S, H, Dh, Dr = 2, 512, 16, 128, 32
D = 2048
DC_SH = 128
BS = B * S
SCALE = (Dh + Dr) ** -0.5


def _dot(a, b):
    return jnp.dot(a, b, preferred_element_type=jnp.float32)


def _dot_t(a, b):
    return lax.dot_general(a, b, (((1,), (1,)), ((), ())),
                           preferred_element_type=jnp.float32)


def _exchange_kv_body(x_ref, wdkv_ref, wuk_ref, wuv_ref, k_ref, v_ref,
                      c_loc, c_rem, wuk_rem, wuv_rem, send_sems, recv_sems):
    my_x = lax.axis_index("x")
    my_y = lax.axis_index("y")
    my_z = lax.axis_index("z")
    peer = (1 - my_x, my_y, my_z)

    c_loc[...] = _dot(x_ref[...], wdkv_ref[...])

    barrier = pltpu.get_barrier_semaphore()
    pl.semaphore_signal(barrier, inc=1, device_id=peer,
                        device_id_type=pl.DeviceIdType.MESH)
    pl.semaphore_wait(barrier, 1)

    copies = []
    for i, (src, dst) in enumerate(
        ((c_loc, c_rem), (wuk_ref, wuk_rem), (wuv_ref, wuv_rem))
    ):
        rdma = pltpu.make_async_remote_copy(
            src_ref=src, dst_ref=dst,
            send_sem=send_sems.at[i], recv_sem=recv_sems.at[i],
            device_id=peer, device_id_type=pl.DeviceIdType.MESH,
        )
        rdma.start()
        copies.append(rdma)
    for rdma in copies:
        rdma.wait()

    k_ref[...] = _dot(c_loc[...], wuk_ref[...]) + _dot(c_rem[...], wuk_rem[...])
    v_ref[...] = _dot(c_loc[...], wuv_ref[...]) + _dot(c_rem[...], wuv_rem[...])


def _proj_body(x_ref, wq_ref, wqr_ref, wkr_ref, q_ref, qr_ref, kr_ref):
    q_ref[...] = _dot(x_ref[...], wq_ref[...])
    qr_ref[...] = _dot(x_ref[...], wqr_ref[...])
    kr_ref[...] = _dot(x_ref[...], wkr_ref[...])


def _attn_body(q_ref, k_ref, v_ref, qr_ref, kr_ref, o_ref):
    s = (_dot_t(q_ref[...], k_ref[...]) + _dot_t(qr_ref[...], kr_ref[...])) * SCALE
    m = jnp.max(s, axis=-1, keepdims=True)
    p = jnp.exp(s - m)
    p = p / jnp.sum(p, axis=-1, keepdims=True)
    o_ref[...] = _dot(p, v_ref[...])


def _out_body(o_ref, wo_ref, out_ref):
    out_ref[...] = _dot(o_ref[...], wo_ref[...])


def kernel(x, Wdkv, Wuk, Wuv, Wq, Wqr, Wkr, Wo):
    x2 = x.reshape(BS, D)

    k, v = pl.pallas_call(
        _exchange_kv_body,
        out_shape=(
            jax.ShapeDtypeStruct((BS, D), jnp.float32),
            jax.ShapeDtypeStruct((BS, D), jnp.float32),
        ),
        in_specs=[pl.BlockSpec(memory_space=pltpu.VMEM)] * 4,
        out_specs=(
            pl.BlockSpec(memory_space=pltpu.VMEM),
            pl.BlockSpec(memory_space=pltpu.VMEM),
        ),
        scratch_shapes=[
            pltpu.VMEM((BS, DC_SH), jnp.float32),
            pltpu.VMEM((BS, DC_SH), jnp.float32),
            pltpu.VMEM((DC_SH, D), jnp.float32),
            pltpu.VMEM((DC_SH, D), jnp.float32),
            pltpu.SemaphoreType.DMA((3,)),
            pltpu.SemaphoreType.DMA((3,)),
        ],
        compiler_params=pltpu.CompilerParams(collective_id=0),
    )(x2, Wdkv, Wuk, Wuv)

    q, qr, kr = pl.pallas_call(
        _proj_body,
        out_shape=(
            jax.ShapeDtypeStruct((BS, D), jnp.float32),
            jax.ShapeDtypeStruct((BS, H * Dr), jnp.float32),
            jax.ShapeDtypeStruct((BS, Dr), jnp.float32),
        ),
        in_specs=[pl.BlockSpec(memory_space=pltpu.VMEM)] * 4,
        out_specs=(pl.BlockSpec(memory_space=pltpu.VMEM),) * 3,
    )(x2, Wq, Wqr, Wkr)

    o = pl.pallas_call(
        _attn_body,
        grid=(B, H),
        out_shape=jax.ShapeDtypeStruct((BS, D), jnp.float32),
        in_specs=[
            pl.BlockSpec((S, Dh), lambda b, h: (b, h)),
            pl.BlockSpec((S, Dh), lambda b, h: (b, h)),
            pl.BlockSpec((S, Dh), lambda b, h: (b, h)),
            pl.BlockSpec((S, Dr), lambda b, h: (b, h)),
            pl.BlockSpec((S, Dr), lambda b, h: (b, 0)),
        ],
        out_specs=pl.BlockSpec((S, Dh), lambda b, h: (b, h)),
    )(q, k, v, qr, kr)

    out = pl.pallas_call(
        _out_body,
        out_shape=jax.ShapeDtypeStruct((BS, D), jnp.float32),
        in_specs=[pl.BlockSpec(memory_space=pltpu.VMEM)] * 2,
        out_specs=pl.BlockSpec(memory_space=pltpu.VMEM),
    )(o, Wo)

    return out.reshape(B, S, D)


# baseline (device time: 116416 ns/iter reference)
import jax
import jax.numpy as jnp
from jax import lax
from jax.experimental import pallas as pl
from jax.experimental.pallas import tpu as pltpu

B, S, H, Dh, Dr = 2, 512, 16, 128, 32
D = 2048
DC_SH = 128
BS = B * S
SCALE = (Dh + Dr) ** -0.5


def _dot(a, b):
    return jnp.dot(a, b, preferred_element_type=jnp.float32)


def _dot_t(a, b):
    return lax.dot_general(a, b, (((1,), (1,)), ((), ())),
                           preferred_element_type=jnp.float32)


def _exchange_kv_body(x_ref, wdkv_ref, wuk_ref, wuv_ref, k_ref, v_ref,
                      c_loc, c_rem, wuk_rem, wuv_rem, send_sems, recv_sems):
    my_x = lax.axis_index("x")
    my_y = lax.axis_index("y")
    my_z = lax.axis_index("z")
    peer = (1 - my_x, my_y, my_z)

    c_loc[...] = _dot(x_ref[...], wdkv_ref[...])

    barrier = pltpu.get_barrier_semaphore()
    pl.semaphore_signal(barrier, inc=1, device_id=peer,
                        device_id_type=pl.DeviceIdType.MESH)
    pl.semaphore_wait(barrier, 1)

    copies = []
    for i, (src, dst) in enumerate(
        ((c_loc, c_rem), (wuk_ref, wuk_rem), (wuv_ref, wuv_rem))
    ):
        rdma = pltpu.make_async_remote_copy(
            src_ref=src, dst_ref=dst,
            send_sem=send_sems.at[i], recv_sem=recv_sems.at[i],
            device_id=peer, device_id_type=pl.DeviceIdType.MESH,
        )
        rdma.start()
        copies.append(rdma)
    for rdma in copies:
        rdma.wait()

    k_ref[...] = _dot(c_loc[...], wuk_ref[...]) + _dot(c_rem[...], wuk_rem[...])
    v_ref[...] = _dot(c_loc[...], wuv_ref[...]) + _dot(c_rem[...], wuv_rem[...])


def _proj_body(x_ref, wq_ref, wqr_ref, wkr_ref, q_ref, qr_ref, kr_ref):
    q_ref[...] = _dot(x_ref[...], wq_ref[...])
    qr_ref[...] = _dot(x_ref[...], wqr_ref[...])
    kr_ref[...] = _dot(x_ref[...], wkr_ref[...])


def _attn_body(q_ref, k_ref, v_ref, qr_ref, kr_ref, o_ref):
    kr = kr_ref[...]
    for h in range(H):
        qh = q_ref[:, h * Dh:(h + 1) * Dh]
        kh = k_ref[:, h * Dh:(h + 1) * Dh]
        qrh = qr_ref[:, h * Dr:(h + 1) * Dr]
        s = (_dot_t(qh, kh) + _dot_t(qrh, kr)) * SCALE
        m = jnp.max(s, axis=-1, keepdims=True)
        p = jnp.exp(s - m)
        p = p / jnp.sum(p, axis=-1, keepdims=True)
        o_ref[:, h * Dh:(h + 1) * Dh] = _dot(p, v_ref[:, h * Dh:(h + 1) * Dh])


def _out_body(o_ref, wo_ref, out_ref):
    out_ref[...] = _dot(o_ref[...], wo_ref[...])


def kernel(x, Wdkv, Wuk, Wuv, Wq, Wqr, Wkr, Wo):
    x2 = x.reshape(BS, D)

    k, v = pl.pallas_call(
        _exchange_kv_body,
        out_shape=(
            jax.ShapeDtypeStruct((BS, D), jnp.float32),
            jax.ShapeDtypeStruct((BS, D), jnp.float32),
        ),
        in_specs=[pl.BlockSpec(memory_space=pltpu.VMEM)] * 4,
        out_specs=(
            pl.BlockSpec(memory_space=pltpu.VMEM),
            pl.BlockSpec(memory_space=pltpu.VMEM),
        ),
        scratch_shapes=[
            pltpu.VMEM((BS, DC_SH), jnp.float32),
            pltpu.VMEM((BS, DC_SH), jnp.float32),
            pltpu.VMEM((DC_SH, D), jnp.float32),
            pltpu.VMEM((DC_SH, D), jnp.float32),
            pltpu.SemaphoreType.DMA((3,)),
            pltpu.SemaphoreType.DMA((3,)),
        ],
        compiler_params=pltpu.CompilerParams(collective_id=0),
    )(x2, Wdkv, Wuk, Wuv)

    q, qr, kr = pl.pallas_call(
        _proj_body,
        out_shape=(
            jax.ShapeDtypeStruct((BS, D), jnp.float32),
            jax.ShapeDtypeStruct((BS, H * Dr), jnp.float32),
            jax.ShapeDtypeStruct((BS, Dr), jnp.float32),
        ),
        in_specs=[pl.BlockSpec(memory_space=pltpu.VMEM)] * 4,
        out_specs=(pl.BlockSpec(memory_space=pltpu.VMEM),) * 3,
    )(x2, Wq, Wqr, Wkr)

    o = pl.pallas_call(
        _attn_body,
        grid=(B,),
        out_shape=jax.ShapeDtypeStruct((BS, D), jnp.float32),
        in_specs=[
            pl.BlockSpec((S, D), lambda b: (b, 0)),
            pl.BlockSpec((S, D), lambda b: (b, 0)),
            pl.BlockSpec((S, D), lambda b: (b, 0)),
            pl.BlockSpec((S, H * Dr), lambda b: (b, 0)),
            pl.BlockSpec((S, Dr), lambda b: (b, 0)),
        ],
        out_specs=pl.BlockSpec((S, D), lambda b: (b, 0)),
    )(q, k, v, qr, kr)

    out = pl.pallas_call(
        _out_body,
        out_shape=jax.ShapeDtypeStruct((BS, D), jnp.float32),
        in_specs=[pl.BlockSpec(memory_space=pltpu.VMEM)] * 2,
        out_specs=pl.BlockSpec(memory_space=pltpu.VMEM),
    )(o, Wo)

    return out.reshape(B, S, D)


# device time: 79713 ns/iter; 1.4604x vs baseline; 1.4604x over previous
import jax
import jax.numpy as jnp
from jax import lax
from jax.experimental import pallas as pl
from jax.experimental.pallas import tpu as pltpu

B, S, H, Dh, Dr = 2, 512, 16, 128, 32
D = 2048
DC_SH = 128
BS = B * S
SCALE = (Dh + Dr) ** -0.5
BF16 = jnp.bfloat16


def _dot(a, b, out=jnp.float32):
    r = jnp.dot(a, b, preferred_element_type=jnp.float32)
    return r if out == jnp.float32 else r.astype(out)


def _dot_t(a, b):
    return lax.dot_general(a, b, (((1,), (1,)), ((), ())),
                           preferred_element_type=jnp.float32)


def _exchange_proj_body(
    x_ref, wdkv_ref, wuk_ref, wuv_ref, wq_ref, wqr_ref, wkr_ref,
    q_ref, qr_ref, kr_ref,
    c_loc, c_rem, wukl, wukr, wuvl, wuvr,
    send_sems, recv_sems,
):
    my_x = lax.axis_index("x")
    my_y = lax.axis_index("y")
    my_z = lax.axis_index("z")
    peer = (1 - my_x, my_y, my_z)

    x = x_ref[...]
    wukl[...] = wuk_ref[...].astype(BF16)
    wuvl[...] = wuv_ref[...].astype(BF16)
    c_loc[...] = _dot(x, wdkv_ref[...], out=BF16)

    barrier = pltpu.get_barrier_semaphore()
    pl.semaphore_signal(barrier, inc=1, device_id=peer,
                        device_id_type=pl.DeviceIdType.MESH)
    pl.semaphore_wait(barrier, 1)

    copies = []
    for i, (src, dst) in enumerate(
        ((c_loc, c_rem), (wukl, wukr), (wuvl, wuvr))
    ):
        rdma = pltpu.make_async_remote_copy(
            src_ref=src, dst_ref=dst,
            send_sem=send_sems.at[i], recv_sem=recv_sems.at[i],
            device_id=peer, device_id_type=pl.DeviceIdType.MESH,
        )
        rdma.start()
        copies.append(rdma)

    q_ref[...] = _dot(x, wq_ref[...], out=BF16)
    qr_ref[...] = _dot(x, wqr_ref[...], out=BF16)
    kr_ref[...] = _dot(x, wkr_ref[...], out=BF16)

    for rdma in copies:
        rdma.wait()


def _attn_body(q_ref, qr_ref, kr_ref, c_loc_ref, c_rem_ref,
               wukl_ref, wukr_ref, wuvl_ref, wuvr_ref, o_ref):
    k = (_dot(c_loc_ref[...], wukl_ref[...])
         + _dot(c_rem_ref[...], wukr_ref[...])).astype(BF16)
    v = (_dot(c_loc_ref[...], wuvl_ref[...])
         + _dot(c_rem_ref[...], wuvr_ref[...])).astype(BF16)
    kr = kr_ref[...]
    for h in range(H):
        qh = q_ref[:, h * Dh:(h + 1) * Dh]
        qrh = qr_ref[:, h * Dr:(h + 1) * Dr]
        s = (_dot_t(qh, k[:, h * Dh:(h + 1) * Dh]) + _dot_t(qrh, kr)) * SCALE
        m = jnp.max(s, axis=-1, keepdims=True)
        p = jnp.exp(s - m)
        p = (p / jnp.sum(p, axis=-1, keepdims=True)).astype(BF16)
        o_ref[:, h * Dh:(h + 1) * Dh] = _dot(p, v[:, h * Dh:(h + 1) * Dh],
                                             out=BF16)


def _out_body(o_ref, wo_ref, out_ref):
    out_ref[...] = _dot(o_ref[...], wo_ref[...].astype(BF16))


def kernel(x, Wdkv, Wuk, Wuv, Wq, Wqr, Wkr, Wo):
    x2 = x.reshape(BS, D)

    q, qr, kr, c_loc, c_rem, wukl, wukr, wuvl, wuvr = pl.pallas_call(
        _exchange_proj_body,
        out_shape=(
            jax.ShapeDtypeStruct((BS, D), BF16),
            jax.ShapeDtypeStruct((BS, H * Dr), BF16),
            jax.ShapeDtypeStruct((BS, Dr), BF16),
            jax.ShapeDtypeStruct((BS, DC_SH), BF16),
            jax.ShapeDtypeStruct((BS, DC_SH), BF16),
            jax.ShapeDtypeStruct((DC_SH, D), BF16),
            jax.ShapeDtypeStruct((DC_SH, D), BF16),
            jax.ShapeDtypeStruct((DC_SH, D), BF16),
            jax.ShapeDtypeStruct((DC_SH, D), BF16),
        ),
        in_specs=[pl.BlockSpec(memory_space=pltpu.VMEM)] * 7,
        out_specs=(pl.BlockSpec(memory_space=pltpu.VMEM),) * 9,
        scratch_shapes=[
            pltpu.SemaphoreType.DMA((3,)),
            pltpu.SemaphoreType.DMA((3,)),
        ],
        compiler_params=pltpu.CompilerParams(collective_id=0),
    )(x2, Wdkv, Wuk, Wuv, Wq, Wqr, Wkr)

    o = pl.pallas_call(
        _attn_body,
        grid=(B,),
        out_shape=jax.ShapeDtypeStruct((BS, D), BF16),
        in_specs=[
            pl.BlockSpec((S, D), lambda b: (b, 0)),
            pl.BlockSpec((S, H * Dr), lambda b: (b, 0)),
            pl.BlockSpec((S, Dr), lambda b: (b, 0)),
            pl.BlockSpec((S, DC_SH), lambda b: (b, 0)),
            pl.BlockSpec((S, DC_SH), lambda b: (b, 0)),
            pl.BlockSpec((DC_SH, D), lambda b: (0, 0)),
            pl.BlockSpec((DC_SH, D), lambda b: (0, 0)),
            pl.BlockSpec((DC_SH, D), lambda b: (0, 0)),
            pl.BlockSpec((DC_SH, D), lambda b: (0, 0)),
        ],
        out_specs=pl.BlockSpec((S, D), lambda b: (b, 0)),
    )(q, qr, kr, c_loc, c_rem, wukl, wukr, wuvl, wuvr)

    out = pl.pallas_call(
        _out_body,
        out_shape=jax.ShapeDtypeStruct((BS, D), jnp.float32),
        in_specs=[pl.BlockSpec(memory_space=pltpu.VMEM)] * 2,
        out_specs=pl.BlockSpec(memory_space=pltpu.VMEM),
    )(o, Wo)

    return out.reshape(B, S, D)


# device time: 69651 ns/iter; 1.6714x vs baseline; 1.1445x over previous
import jax
import jax.numpy as jnp
from jax import lax
from jax.experimental import pallas as pl
from jax.experimental.pallas import tpu as pltpu

B, S, H, Dh, Dr = 2, 512, 16, 128, 32
D = 2048
DC_SH = 128
BS = B * S
SCALE = (Dh + Dr) ** -0.5
BF16 = jnp.bfloat16
NJ = 4
CHUNK = D // NJ


def _dot(a, b, out=jnp.float32):
    r = jnp.dot(a, b, preferred_element_type=jnp.float32)
    return r if out == jnp.float32 else r.astype(out)


def _dot_t(a, b):
    return lax.dot_general(a, b, (((1,), (1,)), ((), ())),
                           preferred_element_type=jnp.float32)


def _exchange_proj_body(
    x_ref, wdkv_ref, wuk_ref, wuv_ref, wq_ref, wqr_ref, wkr_ref,
    q_ref, qr_ref, kr_ref, c_loc, c_rem, wukl, wukr, wuvl, wuvr,
    send_sems, recv_sems,
):
    j = pl.program_id(0)
    my_x = lax.axis_index("x")
    my_y = lax.axis_index("y")
    my_z = lax.axis_index("z")
    peer = (1 - my_x, my_y, my_z)

    def mk(i, src, dst):
        return pltpu.make_async_remote_copy(
            src_ref=src, dst_ref=dst,
            send_sem=send_sems.at[i], recv_sem=recv_sems.at[i],
            device_id=peer, device_id_type=pl.DeviceIdType.MESH,
        )

    @pl.when(j == 0)
    def _():
        barrier = pltpu.get_barrier_semaphore()
        pl.semaphore_signal(barrier, inc=1, device_id=peer,
                            device_id_type=pl.DeviceIdType.MESH)
        pl.semaphore_wait(barrier, 1)
        wukl[...] = wuk_ref[...].astype(BF16)
        wuvl[...] = wuv_ref[...].astype(BF16)
        mk(1, wukl, wukr).start()
        mk(2, wuvl, wuvr).start()
        c_loc[...] = _dot(x_ref[...], wdkv_ref[...], out=BF16)
        mk(0, c_loc, c_rem).start()
        qr_ref[...] = _dot(x_ref[...], wqr_ref[...], out=BF16)
        kr_ref[...] = _dot(x_ref[...], wkr_ref[...], out=BF16)

    q_ref[...] = _dot(x_ref[...], wq_ref[...], out=BF16)

    @pl.when(j == NJ - 1)
    def _():
        for i, (src, dst) in enumerate(
            ((c_loc, c_rem), (wukl, wukr), (wuvl, wuvr))
        ):
            mk(i, src, dst).wait()


def _attn_body(q_ref, qr_ref, kr_ref, c_loc_ref, c_rem_ref,
               wukl_ref, wukr_ref, wuvl_ref, wuvr_ref, o_ref):
    cl, cr = c_loc_ref[...], c_rem_ref[...]

    def _assemble(wl_ref, wr_ref):
        chunks = []
        for c0 in range(0, D, CHUNK):
            sl = slice(c0, c0 + CHUNK)
            chunks.append((_dot(cl, wl_ref[:, sl])
                           + _dot(cr, wr_ref[:, sl])).astype(BF16))
        return jnp.concatenate(chunks, axis=1)

    k = _assemble(wukl_ref, wukr_ref)
    v = _assemble(wuvl_ref, wuvr_ref)
    kr = kr_ref[...]
    o_cols = []
    for h in range(H):
        qh = q_ref[:, h * Dh:(h + 1) * Dh]
        qrh = qr_ref[:, h * Dr:(h + 1) * Dr]
        s = (_dot_t(qh, k[:, h * Dh:(h + 1) * Dh]) + _dot_t(qrh, kr)) * SCALE
        p = jnp.exp(s)
        recip = 1.0 / jnp.sum(p, axis=-1, keepdims=True)
        oh = _dot(p.astype(BF16), v[:, h * Dh:(h + 1) * Dh]) * recip
        o_cols.append(oh.astype(BF16))
    o_ref[...] = jnp.concatenate(o_cols, axis=1)


def _out_body(o_ref, wo_ref, out_ref):
    out_ref[...] = _dot(o_ref[...], wo_ref[...].astype(BF16))


def kernel(x, Wdkv, Wuk, Wuv, Wq, Wqr, Wkr, Wo):
    x2 = x.reshape(BS, D)

    q, qr, kr, c_loc, c_rem, wukl, wukr, wuvl, wuvr = pl.pallas_call(
        _exchange_proj_body,
        grid=(NJ,),
        out_shape=(
            jax.ShapeDtypeStruct((BS, D), BF16),
            jax.ShapeDtypeStruct((BS, H * Dr), BF16),
            jax.ShapeDtypeStruct((BS, Dr), BF16),
            jax.ShapeDtypeStruct((BS, DC_SH), BF16),
            jax.ShapeDtypeStruct((BS, DC_SH), BF16),
            jax.ShapeDtypeStruct((DC_SH, D), BF16),
            jax.ShapeDtypeStruct((DC_SH, D), BF16),
            jax.ShapeDtypeStruct((DC_SH, D), BF16),
            jax.ShapeDtypeStruct((DC_SH, D), BF16),
        ),
        in_specs=[
            pl.BlockSpec((BS, D), lambda j: (0, 0)),
            pl.BlockSpec((D, DC_SH), lambda j: (0, 0)),
            pl.BlockSpec((DC_SH, D), lambda j: (0, 0)),
            pl.BlockSpec((DC_SH, D), lambda j: (0, 0)),
            pl.BlockSpec((D, CHUNK), lambda j: (0, j)),
            pl.BlockSpec((D, H * Dr), lambda j: (0, 0)),
            pl.BlockSpec((D, Dr), lambda j: (0, 0)),
        ],
        out_specs=(
            pl.BlockSpec((BS, CHUNK), lambda j: (0, j)),
            pl.BlockSpec((BS, H * Dr), lambda j: (0, 0)),
            pl.BlockSpec((BS, Dr), lambda j: (0, 0)),
            pl.BlockSpec((BS, DC_SH), lambda j: (0, 0)),
            pl.BlockSpec((BS, DC_SH), lambda j: (0, 0)),
            pl.BlockSpec((DC_SH, D), lambda j: (0, 0)),
            pl.BlockSpec((DC_SH, D), lambda j: (0, 0)),
            pl.BlockSpec((DC_SH, D), lambda j: (0, 0)),
            pl.BlockSpec((DC_SH, D), lambda j: (0, 0)),
        ),
        scratch_shapes=[
            pltpu.SemaphoreType.DMA((3,)),
            pltpu.SemaphoreType.DMA((3,)),
        ],
        compiler_params=pltpu.CompilerParams(collective_id=0),
    )(x2, Wdkv, Wuk, Wuv, Wq, Wqr, Wkr)

    o = pl.pallas_call(
        _attn_body,
        grid=(B,),
        out_shape=jax.ShapeDtypeStruct((BS, D), BF16),
        in_specs=[
            pl.BlockSpec((S, D), lambda b: (b, 0)),
            pl.BlockSpec((S, H * Dr), lambda b: (b, 0)),
            pl.BlockSpec((S, Dr), lambda b: (b, 0)),
            pl.BlockSpec((S, DC_SH), lambda b: (b, 0)),
            pl.BlockSpec((S, DC_SH), lambda b: (b, 0)),
            pl.BlockSpec((DC_SH, D), lambda b: (0, 0)),
            pl.BlockSpec((DC_SH, D), lambda b: (0, 0)),
            pl.BlockSpec((DC_SH, D), lambda b: (0, 0)),
            pl.BlockSpec((DC_SH, D), lambda b: (0, 0)),
        ],
        out_specs=pl.BlockSpec((S, D), lambda b: (b, 0)),
    )(q, qr, kr, c_loc, c_rem, wukl, wukr, wuvl, wuvr)

    out = pl.pallas_call(
        _out_body,
        grid=(NJ,),
        out_shape=jax.ShapeDtypeStruct((BS, D), jnp.float32),
        in_specs=[
            pl.BlockSpec((BS, D), lambda j: (0, 0)),
            pl.BlockSpec((D, CHUNK), lambda j: (0, j)),
        ],
        out_specs=pl.BlockSpec((BS, CHUNK), lambda j: (0, j)),
    )(o, Wo)

    return out.reshape(B, S, D)
